# baseline (speedup 1.0000x reference)
import functools

import jax
import jax.numpy as jnp
from jax import lax
from jax.experimental import pallas as pl
from jax.experimental.pallas import tpu as pltpu
from jax.experimental.pallas import tpu_sc as plsc

N = 200000
C = 64
SCALE = 4
GLOBAL_SIZE = 64
N_GLOBAL = GLOBAL_SIZE ** 3

NC = 2
NS = 16
NW = NC * NS

QB = 128
TB = 1024
H = TB // 2
NB = (N + TB - 1) // TB
NP = NB * TB
G = 2 * NB


QC = 256
G4 = NP // QC


def _gather_body(cx_hbm, cy_hbm, cz_hbm, gf_hbm, g2_hbm,
                 cx0, cy0, cz0, cx1, cy1, cz1,
                 i00, i01, i10, i11,
                 d00, d01, d10, d11,
                 csem0, csem1, gsem0, gsem1, ssem0, ssem1):
    wid = lax.axis_index("s") * NC + lax.axis_index("c")
    cnt = (G4 + NW - 1 - wid) // NW

    sets = (
        ((cx0, cy0, cz0), (i00, i01), (d00, d01), csem0, gsem0, ssem0),
        ((cx1, cy1, cz1), (i10, i11), (d10, d11), csem1, gsem1, ssem1),
    )

    def coords_of(c):
        return (c >> 2) * TB + (c & 3) * QC

    def g2_row(c):
        return (c >> 2) * H + (c & 1) * QC

    def start_coords(c, s):
        (cxv, cyv, czv), _, _, csem, _, _ = sets[s]
        base = jnp.minimum(coords_of(c), NP - QC)
        pltpu.make_async_copy(cx_hbm.at[pl.ds(base, QC)], cxv, csem).start()
        pltpu.make_async_copy(cy_hbm.at[pl.ds(base, QC)], cyv, csem).start()
        pltpu.make_async_copy(cz_hbm.at[pl.ds(base, QC)], czv, csem).start()

    def phase(j, c, s):
        (cxv, cyv, czv), idxs, dsts, csem, gsem, ssem = sets[s]
        base = coords_of(c)
        half = (c >> 1) & 1
        grow = g2_row(c)

        pltpu.make_async_copy(cx_hbm.at[pl.ds(base, QC)], cxv, csem).wait()
        pltpu.make_async_copy(cy_hbm.at[pl.ds(base, QC)], cyv, csem).wait()
        pltpu.make_async_copy(cz_hbm.at[pl.ds(base, QC)], czv, csem).wait()

        for t in range(QC // 16):
            sl = pl.ds(t * 16, 16)
            x = jnp.clip(cxv[sl] >> 2, 0, GLOBAL_SIZE - 1)
            y = jnp.clip(cyv[sl] >> 2, 0, GLOBAL_SIZE - 1)
            z = jnp.clip(czv[sl] >> 2, 0, GLOBAL_SIZE - 1)
            flat = (x * (GLOBAL_SIZE * GLOBAL_SIZE) + y * GLOBAL_SIZE) + z
            idxs[t // 8][pl.ds((t % 8) * 16, 16)] = flat

        gcps = [pltpu.async_copy(gf_hbm.at[idxs[q]], dsts[q], gsem)
                for q in range(2)]

        start_coords(c + NW, 1 - s)

        @pl.when(j > 1)
        def _():
            for q in range(2):
                pltpu.make_async_copy(
                    dsts[q].at[:, pl.ds(0, C)],
                    g2_hbm.at[pl.ds(grow + q * QB, QB), pl.ds(half * C, C)],
                    ssem).wait()

        for cp in gcps:
            cp.wait()

        for q in range(2):
            pltpu.make_async_copy(
                dsts[q].at[:, pl.ds(0, C)],
                g2_hbm.at[pl.ds(grow + q * QB, QB), pl.ds(half * C, C)],
                ssem).start()

    start_coords(wid, 0)

    def pair_body(j, carry):
        phase(2 * j, wid + (2 * j) * NW, 0)
        phase(2 * j + 1, wid + (2 * j + 1) * NW, 1)
        return carry

    lax.fori_loop(0, cnt // 2, pair_body, 0)

    @pl.when(cnt % 2 == 1)
    def _():
        phase(cnt - 1, wid + (cnt - 1) * NW, 0)

    for s in (0, 1):
        @pl.when(cnt % 2 == s)
        def _(s=s):
            (cxv, cyv, czv), _, _, csem, _, _ = sets[s]
            pltpu.make_async_copy(cx_hbm.at[pl.ds(0, QC)], cxv, csem).wait()
            pltpu.make_async_copy(cy_hbm.at[pl.ds(0, QC)], cyv, csem).wait()
            pltpu.make_async_copy(cz_hbm.at[pl.ds(0, QC)], czv, csem).wait()

    for s in (0, 1):
        _, _, dsts, _, _, ssem = sets[s]
        for q in range(2):
            pltpu.make_async_copy(
                dsts[q].at[:, pl.ds(0, C)],
                g2_hbm.at[pl.ds(q * QB, QB), pl.ds(0, C)],
                ssem).wait()


FB = 4
FW = FB * TB
EB = 4096


def _expand_body(gft_ref, gfp_ref):
    gfp_ref[:, 0:C] = gft_ref[...].T
    gfp_ref[:, C:2 * C] = jnp.zeros((EB, C), jnp.float32)


def _fuse_body(lf_ref, g2_ref, out_ref):
    for b in range(FB):
        lft = lf_ref[:, pl.ds(b * TB, TB)]
        g2t = g2_ref[pl.ds(b * H, H), :].T
        out_ref[:, pl.ds(b * TB, H)] = lft[:, 0:H] + g2t[0:C, :]
        out_ref[:, pl.ds(b * TB + H, H)] = lft[:, H:TB] + g2t[C:2 * C, :]


@jax.jit
def _fusion(cx, cy, cz, lft, gft):
    gfp = pl.pallas_call(
        _expand_body,
        grid=(N_GLOBAL // EB,),
        in_specs=[pl.BlockSpec((C, EB), lambda i: (0, i))],
        out_specs=pl.BlockSpec((EB, 2 * C), lambda i: (i, 0)),
        out_shape=jax.ShapeDtypeStruct((N_GLOBAL, 2 * C), jnp.float32),
    )(gft)

    mesh = plsc.VectorSubcoreMesh(core_axis_name="c", subcore_axis_name="s")
    g2 = pl.kernel(
        _gather_body,
        out_type=jax.ShapeDtypeStruct((NB * H, 2 * C), jnp.float32),
        mesh=mesh,
        scratch_types=(
            [pltpu.VMEM((QC,), jnp.int32)] * 6
            + [pltpu.VMEM((QB,), jnp.int32)] * 4
            + [pltpu.VMEM((QB, 2 * C), jnp.float32)] * 4
            + [pltpu.SemaphoreType.DMA] * 6
        ),
        compiler_params=pltpu.CompilerParams(
            use_tc_tiling_on_sc=False, needs_layout_passes=False),
    )(cx, cy, cz, gfp)

    out_t = pl.pallas_call(
        _fuse_body,
        grid=(NB // FB,),
        in_specs=[
            pl.BlockSpec((C, FW), lambda i: (0, i)),
            pl.BlockSpec((FB * H, 2 * C), lambda i: (i, 0)),
        ],
        out_specs=pl.BlockSpec((C, FW), lambda i: (0, i)),
        out_shape=jax.ShapeDtypeStruct((C, N), jnp.float32),
    )(lft, g2)
    return out_t


def kernel(local_features, local_coords, local_base, global_features, global_base):
    adj = (local_coords.astype(jnp.int32)
           + local_base.astype(jnp.int32)[None, :]
           - SCALE * global_base.astype(jnp.int32)[None, :])
    pad = (0, NP - N)
    cx = jnp.pad(adj[:, 0], pad)
    cy = jnp.pad(adj[:, 1], pad)
    cz = jnp.pad(adj[:, 2], pad)
    out_t = _fusion(cx, cy, cz, local_features.T, global_features.T)
    return out_t.T


# speedup vs baseline: 1.0574x; 1.0574x over previous
import functools

import jax
import jax.numpy as jnp
from jax import lax
from jax.experimental import pallas as pl
from jax.experimental.pallas import tpu as pltpu
from jax.experimental.pallas import tpu_sc as plsc

N = 200000
C = 64
SCALE = 4
GLOBAL_SIZE = 64
N_GLOBAL = GLOBAL_SIZE ** 3

NC = 2
NS = 16
NW = NC * NS

QB = 128
TB = 1024
H = TB // 2
NB = (N + TB - 1) // TB
NP = NB * TB
G = 2 * NB


QC = 256
NBA = 96


def _gather_body(b0, g4p, cx_hbm, cy_hbm, cz_hbm, gf_hbm, g2_hbm,
                 cx0, cy0, cz0, cx1, cy1, cz1,
                 i00, i01, i10, i11,
                 d00, d01, d10, d11,
                 csem0, csem1, gsem0, gsem1, ssem0, ssem1):
    wid = lax.axis_index("s") * NC + lax.axis_index("c")
    cnt = (g4p + NW - 1 - wid) // NW

    sets = (
        ((cx0, cy0, cz0), (i00, i01), (d00, d01), csem0, gsem0, ssem0),
        ((cx1, cy1, cz1), (i10, i11), (d10, d11), csem1, gsem1, ssem1),
    )

    def coords_of(c):
        return (b0 + (c >> 2)) * TB + (c & 3) * QC

    def g2_row(c):
        return (c >> 2) * H + (c & 1) * QC

    def start_coords(c, s):
        (cxv, cyv, czv), _, _, csem, _, _ = sets[s]
        base = jnp.minimum(coords_of(c), NP - QC)
        pltpu.make_async_copy(cx_hbm.at[pl.ds(base, QC)], cxv, csem).start()
        pltpu.make_async_copy(cy_hbm.at[pl.ds(base, QC)], cyv, csem).start()
        pltpu.make_async_copy(cz_hbm.at[pl.ds(base, QC)], czv, csem).start()

    def phase(j, c, s):
        (cxv, cyv, czv), idxs, dsts, csem, gsem, ssem = sets[s]
        base = coords_of(c)
        half = (c >> 1) & 1
        grow = g2_row(c)

        pltpu.make_async_copy(cx_hbm.at[pl.ds(base, QC)], cxv, csem).wait()
        pltpu.make_async_copy(cy_hbm.at[pl.ds(base, QC)], cyv, csem).wait()
        pltpu.make_async_copy(cz_hbm.at[pl.ds(base, QC)], czv, csem).wait()

        for t in range(QC // 16):
            sl = pl.ds(t * 16, 16)
            x = jnp.clip(cxv[sl] >> 2, 0, GLOBAL_SIZE - 1)
            y = jnp.clip(cyv[sl] >> 2, 0, GLOBAL_SIZE - 1)
            z = jnp.clip(czv[sl] >> 2, 0, GLOBAL_SIZE - 1)
            flat = (x * (GLOBAL_SIZE * GLOBAL_SIZE) + y * GLOBAL_SIZE) + z
            idxs[t // 8][pl.ds((t % 8) * 16, 16)] = flat

        gcps = [pltpu.async_copy(gf_hbm.at[idxs[q]], dsts[q], gsem)
                for q in range(2)]

        start_coords(c + NW, 1 - s)

        @pl.when(j > 1)
        def _():
            for q in range(2):
                pltpu.make_async_copy(
                    dsts[q].at[:, pl.ds(0, C)],
                    g2_hbm.at[pl.ds(grow + q * QB, QB), pl.ds(half * C, C)],
                    ssem).wait()

        for cp in gcps:
            cp.wait()

        for q in range(2):
            pltpu.make_async_copy(
                dsts[q].at[:, pl.ds(0, C)],
                g2_hbm.at[pl.ds(grow + q * QB, QB), pl.ds(half * C, C)],
                ssem).start()

    start_coords(wid, 0)

    def pair_body(j, carry):
        phase(2 * j, wid + (2 * j) * NW, 0)
        phase(2 * j + 1, wid + (2 * j + 1) * NW, 1)
        return carry

    lax.fori_loop(0, cnt // 2, pair_body, 0)

    @pl.when(cnt % 2 == 1)
    def _():
        phase(cnt - 1, wid + (cnt - 1) * NW, 0)

    for s in (0, 1):
        @pl.when(cnt % 2 == s)
        def _(s=s):
            (cxv, cyv, czv), _, _, csem, _, _ = sets[s]
            pltpu.make_async_copy(cx_hbm.at[pl.ds(0, QC)], cxv, csem).wait()
            pltpu.make_async_copy(cy_hbm.at[pl.ds(0, QC)], cyv, csem).wait()
            pltpu.make_async_copy(cz_hbm.at[pl.ds(0, QC)], czv, csem).wait()

    for s in (0, 1):
        _, _, dsts, _, _, ssem = sets[s]
        for q in range(2):
            pltpu.make_async_copy(
                dsts[q].at[:, pl.ds(0, C)],
                g2_hbm.at[pl.ds(q * QB, QB), pl.ds(0, C)],
                ssem).wait()


FB = 4
FW = FB * TB
EB = 4096


def _expand_body(gft_ref, gfp_ref):
    gfp_ref[:, 0:C] = gft_ref[...].T
    gfp_ref[:, C:2 * C] = jnp.zeros((EB, C), jnp.float32)


def _fuse_body2(lf_ref, g2_ref, prev_ref, out_ref):
    del prev_ref
    _fuse_body(lf_ref, g2_ref, out_ref)


def _fuse_body(lf_ref, g2_ref, out_ref):
    for b in range(FB):
        lft = lf_ref[:, pl.ds(b * TB, TB)]
        g2t = g2_ref[pl.ds(b * H, H), :].T
        out_ref[:, pl.ds(b * TB, H)] = lft[:, 0:H] + g2t[0:C, :]
        out_ref[:, pl.ds(b * TB + H, H)] = lft[:, H:TB] + g2t[C:2 * C, :]


@jax.jit
def _fusion(cx, cy, cz, lft, gft):
    gfp = pl.pallas_call(
        _expand_body,
        grid=(N_GLOBAL // EB,),
        in_specs=[pl.BlockSpec((C, EB), lambda i: (0, i))],
        out_specs=pl.BlockSpec((EB, 2 * C), lambda i: (i, 0)),
        out_shape=jax.ShapeDtypeStruct((N_GLOBAL, 2 * C), jnp.float32),
    )(gft)

    mesh = plsc.VectorSubcoreMesh(core_axis_name="c", subcore_axis_name="s")

    def sc_gather(b0, nblocks):
        return pl.kernel(
            functools.partial(_gather_body, b0, 4 * nblocks),
            out_type=jax.ShapeDtypeStruct((nblocks * H, 2 * C), jnp.float32),
            mesh=mesh,
            scratch_types=(
                [pltpu.VMEM((QC,), jnp.int32)] * 6
                + [pltpu.VMEM((QB,), jnp.int32)] * 4
                + [pltpu.VMEM((QB, 2 * C), jnp.float32)] * 4
                + [pltpu.SemaphoreType.DMA] * 6
            ),
            compiler_params=pltpu.CompilerParams(
                use_tc_tiling_on_sc=False, needs_layout_passes=False),
        )(cx, cy, cz, gfp)

    g2a = sc_gather(0, NBA)
    g2b = sc_gather(NBA, NB - NBA)

    out_a = pl.pallas_call(
        _fuse_body,
        grid=(NBA // FB,),
        in_specs=[
            pl.BlockSpec((C, FW), lambda i: (0, i)),
            pl.BlockSpec((FB * H, 2 * C), lambda i: (i, 0)),
        ],
        out_specs=pl.BlockSpec((C, FW), lambda i: (0, i)),
        out_shape=jax.ShapeDtypeStruct((C, N), jnp.float32),
    )(lft, g2a)

    off = NBA // FB
    out_t = pl.pallas_call(
        _fuse_body2,
        grid=((NB - NBA) // FB,),
        in_specs=[
            pl.BlockSpec((C, FW), lambda i, off=off: (0, i + off)),
            pl.BlockSpec((FB * H, 2 * C), lambda i: (i, 0)),
            pl.BlockSpec(memory_space=pl.ANY),
        ],
        out_specs=pl.BlockSpec((C, FW), lambda i, off=off: (0, i + off)),
        out_shape=jax.ShapeDtypeStruct((C, N), jnp.float32),
        input_output_aliases={2: 0},
    )(lft, g2b, out_a)
    return out_t


def kernel(local_features, local_coords, local_base, global_features, global_base):
    adj = (local_coords.astype(jnp.int32)
           + local_base.astype(jnp.int32)[None, :]
           - SCALE * global_base.astype(jnp.int32)[None, :])
    pad = (0, NP - N)
    cx = jnp.pad(adj[:, 0], pad)
    cy = jnp.pad(adj[:, 1], pad)
    cz = jnp.pad(adj[:, 2], pad)
    out_t = _fusion(cx, cy, cz, local_features.T, global_features.T)
    return out_t.T


# speedup vs baseline: 1.0971x; 1.0375x over previous
import functools

import jax
import jax.numpy as jnp
from jax import lax
from jax.experimental import pallas as pl
from jax.experimental.pallas import tpu as pltpu
from jax.experimental.pallas import tpu_sc as plsc

N = 200000
C = 64
SCALE = 4
GLOBAL_SIZE = 64
N_GLOBAL = GLOBAL_SIZE ** 3

NC = 2
NS = 16
NW = NC * NS

QB = 128
TB = 1024
H = TB // 2
NB = (N + TB - 1) // TB
NP = NB * TB
G = 2 * NB


QC = 256
NBA = 96


def _gather_body(b0, g4p, cx_hbm, cy_hbm, cz_hbm, gf_hbm, g2_hbm,
                 cx0, cy0, cz0, cx1, cy1, cz1,
                 i00, i01, i10, i11,
                 d00, d01, d10, d11,
                 csem0, csem1, gsem0, gsem1, ssem0, ssem1):
    wid = lax.axis_index("s") * NC + lax.axis_index("c")
    cnt = (g4p + NW - 1 - wid) // NW

    sets = (
        ((cx0, cy0, cz0), (i00, i01), (d00, d01), csem0, gsem0, ssem0),
        ((cx1, cy1, cz1), (i10, i11), (d10, d11), csem1, gsem1, ssem1),
    )

    def coords_of(c):
        return (b0 + (c >> 2)) * TB + (c & 3) * QC

    def g2_row(c):
        return (c >> 2) * H + (c & 1) * QC

    def start_coords(c, s):
        (cxv, cyv, czv), _, _, csem, _, _ = sets[s]
        base = jnp.minimum(coords_of(c), NP - QC)
        pltpu.make_async_copy(cx_hbm.at[pl.ds(base, QC)], cxv, csem).start()
        pltpu.make_async_copy(cy_hbm.at[pl.ds(base, QC)], cyv, csem).start()
        pltpu.make_async_copy(cz_hbm.at[pl.ds(base, QC)], czv, csem).start()

    def phase(j, c, s):
        (cxv, cyv, czv), idxs, dsts, csem, gsem, ssem = sets[s]
        base = coords_of(c)
        half = (c >> 1) & 1
        grow = g2_row(c)

        pltpu.make_async_copy(cx_hbm.at[pl.ds(base, QC)], cxv, csem).wait()
        pltpu.make_async_copy(cy_hbm.at[pl.ds(base, QC)], cyv, csem).wait()
        pltpu.make_async_copy(cz_hbm.at[pl.ds(base, QC)], czv, csem).wait()

        for t in range(QC // 16):
            sl = pl.ds(t * 16, 16)
            x = jnp.clip(cxv[sl] >> 2, 0, GLOBAL_SIZE - 1)
            y = jnp.clip(cyv[sl] >> 2, 0, GLOBAL_SIZE - 1)
            z = jnp.clip(czv[sl] >> 2, 0, GLOBAL_SIZE - 1)
            flat = (x * (GLOBAL_SIZE * GLOBAL_SIZE) + y * GLOBAL_SIZE) + z
            idxs[t // 8][pl.ds((t % 8) * 16, 16)] = flat

        gcps = [pltpu.async_copy(gf_hbm.at[idxs[q]], dsts[q], gsem)
                for q in range(2)]

        start_coords(c + NW, 1 - s)

        @pl.when(j > 1)
        def _():
            for q in range(2):
                pltpu.make_async_copy(
                    dsts[q].at[:, pl.ds(0, C)],
                    g2_hbm.at[pl.ds(grow + q * QB, QB), pl.ds(half * C, C)],
                    ssem).wait()

        for cp in gcps:
            cp.wait()

        for q in range(2):
            pltpu.make_async_copy(
                dsts[q].at[:, pl.ds(0, C)],
                g2_hbm.at[pl.ds(grow + q * QB, QB), pl.ds(half * C, C)],
                ssem).start()

    start_coords(wid, 0)

    def pair_body(j, carry):
        phase(2 * j, wid + (2 * j) * NW, 0)
        phase(2 * j + 1, wid + (2 * j + 1) * NW, 1)
        return carry

    lax.fori_loop(0, cnt // 2, pair_body, 0)

    @pl.when(cnt % 2 == 1)
    def _():
        phase(cnt - 1, wid + (cnt - 1) * NW, 0)

    for s in (0, 1):
        @pl.when(cnt % 2 == s)
        def _(s=s):
            (cxv, cyv, czv), _, _, csem, _, _ = sets[s]
            pltpu.make_async_copy(cx_hbm.at[pl.ds(0, QC)], cxv, csem).wait()
            pltpu.make_async_copy(cy_hbm.at[pl.ds(0, QC)], cyv, csem).wait()
            pltpu.make_async_copy(cz_hbm.at[pl.ds(0, QC)], czv, csem).wait()

    for s in (0, 1):
        _, _, dsts, _, _, ssem = sets[s]
        for q in range(2):
            pltpu.make_async_copy(
                dsts[q].at[:, pl.ds(0, C)],
                g2_hbm.at[pl.ds(q * QB, QB), pl.ds(0, C)],
                ssem).wait()


FB = 4
FW = FB * TB
EB = 8192
PARTS = (48, 48, 48, 52)


def _expand_body(gft_ref, gfp_ref):
    gfp_ref[:, 0:C] = gft_ref[...].T
    gfp_ref[:, C:2 * C] = jnp.zeros((EB, C), jnp.float32)


def _fuse_body2(lf_ref, g2_ref, prev_ref, out_ref):
    del prev_ref
    _fuse_body(lf_ref, g2_ref, out_ref)


def _fuse_body(lf_ref, g2_ref, out_ref):
    for b in range(FB):
        lft = lf_ref[:, pl.ds(b * TB, TB)]
        g2t = g2_ref[pl.ds(b * H, H), :].T
        out_ref[:, pl.ds(b * TB, H)] = lft[:, 0:H] + g2t[0:C, :]
        out_ref[:, pl.ds(b * TB + H, H)] = lft[:, H:TB] + g2t[C:2 * C, :]


@jax.jit
def _fusion(cx, cy, cz, lft, gft):
    gfp = pl.pallas_call(
        _expand_body,
        grid=(N_GLOBAL // EB,),
        in_specs=[pl.BlockSpec((C, EB), lambda i: (0, i))],
        out_specs=pl.BlockSpec((EB, 2 * C), lambda i: (i, 0)),
        out_shape=jax.ShapeDtypeStruct((N_GLOBAL, 2 * C), jnp.float32),
    )(gft)

    mesh = plsc.VectorSubcoreMesh(core_axis_name="c", subcore_axis_name="s")

    def sc_gather(b0, nblocks):
        return pl.kernel(
            functools.partial(_gather_body, b0, 4 * nblocks),
            out_type=jax.ShapeDtypeStruct((nblocks * H, 2 * C), jnp.float32),
            mesh=mesh,
            scratch_types=(
                [pltpu.VMEM((QC,), jnp.int32)] * 6
                + [pltpu.VMEM((QB,), jnp.int32)] * 4
                + [pltpu.VMEM((QB, 2 * C), jnp.float32)] * 4
                + [pltpu.SemaphoreType.DMA] * 6
            ),
            compiler_params=pltpu.CompilerParams(
                use_tc_tiling_on_sc=False, needs_layout_passes=False),
        )(cx, cy, cz, gfp)

    g2s = []
    b0 = 0
    for nb in PARTS:
        g2s.append(sc_gather(b0, nb))
        b0 += nb

    out_t = None
    b0 = 0
    for p, nb in enumerate(PARTS):
        off = b0 // FB
        lf_spec = pl.BlockSpec((C, FW), lambda i, off=off: (0, i + off))
        g2_spec = pl.BlockSpec((FB * H, 2 * C), lambda i: (i, 0))
        out_spec = pl.BlockSpec((C, FW), lambda i, off=off: (0, i + off))
        out_shape = jax.ShapeDtypeStruct((C, N), jnp.float32)
        if p == 0:
            out_t = pl.pallas_call(
                _fuse_body,
                grid=(nb // FB,),
                in_specs=[lf_spec, g2_spec],
                out_specs=out_spec,
                out_shape=out_shape,
            )(lft, g2s[p])
        else:
            out_t = pl.pallas_call(
                _fuse_body2,
                grid=(nb // FB,),
                in_specs=[lf_spec, g2_spec,
                          pl.BlockSpec(memory_space=pl.ANY)],
                out_specs=out_spec,
                out_shape=out_shape,
                input_output_aliases={2: 0},
            )(lft, g2s[p], out_t)
        b0 += nb
    return out_t


def kernel(local_features, local_coords, local_base, global_features, global_base):
    adj = (local_coords.astype(jnp.int32)
           + local_base.astype(jnp.int32)[None, :]
           - SCALE * global_base.astype(jnp.int32)[None, :])
    pad = (0, NP - N)
    cx = jnp.pad(adj[:, 0], pad)
    cy = jnp.pad(adj[:, 1], pad)
    cz = jnp.pad(adj[:, 2], pad)
    out_t = _fusion(cx, cy, cz, local_features.T, global_features.T)
    return out_t.T


# speedup vs baseline: 1.1099x; 1.0117x over previous
import functools

import jax
import jax.numpy as jnp
from jax import lax
from jax.experimental import pallas as pl
from jax.experimental.pallas import tpu as pltpu
from jax.experimental.pallas import tpu_sc as plsc

N = 200000
C = 64
SCALE = 4
GLOBAL_SIZE = 64
N_GLOBAL = GLOBAL_SIZE ** 3

NC = 2
NS = 16
NW = NC * NS

QB = 128
TB = 1024
H = TB // 2
NB = (N + TB - 1) // TB
NP = NB * TB
G = 2 * NB


QC = 256
NBA = 96


def _gather_body(b0, g4p, cx_hbm, cy_hbm, cz_hbm, gf_hbm, g2_hbm,
                 cx0, cy0, cz0, cx1, cy1, cz1,
                 i00, i01, i10, i11,
                 d00, d01, d10, d11,
                 csem0, csem1, gsem0, gsem1, ssem0, ssem1):
    wid = lax.axis_index("s") * NC + lax.axis_index("c")
    cnt = (g4p + NW - 1 - wid) // NW

    sets = (
        ((cx0, cy0, cz0), (i00, i01), (d00, d01), csem0, gsem0, ssem0),
        ((cx1, cy1, cz1), (i10, i11), (d10, d11), csem1, gsem1, ssem1),
    )

    def coords_of(c):
        return (b0 + (c >> 2)) * TB + (c & 3) * QC

    def g2_row(c):
        return (c >> 2) * H + (c & 1) * QC

    def start_coords(c, s):
        (cxv, cyv, czv), _, _, csem, _, _ = sets[s]
        base = jnp.minimum(coords_of(c), NP - QC)
        pltpu.make_async_copy(cx_hbm.at[pl.ds(base, QC)], cxv, csem).start()
        pltpu.make_async_copy(cy_hbm.at[pl.ds(base, QC)], cyv, csem).start()
        pltpu.make_async_copy(cz_hbm.at[pl.ds(base, QC)], czv, csem).start()

    def phase(j, c, s):
        (cxv, cyv, czv), idxs, dsts, csem, gsem, ssem = sets[s]
        base = coords_of(c)
        half = (c >> 1) & 1
        grow = g2_row(c)

        pltpu.make_async_copy(cx_hbm.at[pl.ds(base, QC)], cxv, csem).wait()
        pltpu.make_async_copy(cy_hbm.at[pl.ds(base, QC)], cyv, csem).wait()
        pltpu.make_async_copy(cz_hbm.at[pl.ds(base, QC)], czv, csem).wait()

        for t in range(QC // 16):
            sl = pl.ds(t * 16, 16)
            x = jnp.clip(cxv[sl] >> 2, 0, GLOBAL_SIZE - 1)
            y = jnp.clip(cyv[sl] >> 2, 0, GLOBAL_SIZE - 1)
            z = jnp.clip(czv[sl] >> 2, 0, GLOBAL_SIZE - 1)
            flat = (x * (GLOBAL_SIZE * GLOBAL_SIZE) + y * GLOBAL_SIZE) + z
            idxs[t // 8][pl.ds((t % 8) * 16, 16)] = flat

        gcps = [pltpu.async_copy(gf_hbm.at[idxs[q]], dsts[q], gsem)
                for q in range(2)]

        start_coords(c + NW, 1 - s)

        @pl.when(j > 1)
        def _():
            for q in range(2):
                pltpu.make_async_copy(
                    dsts[q].at[:, pl.ds(0, C)],
                    g2_hbm.at[pl.ds(grow + q * QB, QB), pl.ds(half * C, C)],
                    ssem).wait()

        for cp in gcps:
            cp.wait()

        for q in range(2):
            pltpu.make_async_copy(
                dsts[q].at[:, pl.ds(0, C)],
                g2_hbm.at[pl.ds(grow + q * QB, QB), pl.ds(half * C, C)],
                ssem).start()

    start_coords(wid, 0)

    def pair_body(j, carry):
        phase(2 * j, wid + (2 * j) * NW, 0)
        phase(2 * j + 1, wid + (2 * j + 1) * NW, 1)
        return carry

    lax.fori_loop(0, cnt // 2, pair_body, 0)

    @pl.when(cnt % 2 == 1)
    def _():
        phase(cnt - 1, wid + (cnt - 1) * NW, 0)

    for s in (0, 1):
        @pl.when(cnt % 2 == s)
        def _(s=s):
            (cxv, cyv, czv), _, _, csem, _, _ = sets[s]
            pltpu.make_async_copy(cx_hbm.at[pl.ds(0, QC)], cxv, csem).wait()
            pltpu.make_async_copy(cy_hbm.at[pl.ds(0, QC)], cyv, csem).wait()
            pltpu.make_async_copy(cz_hbm.at[pl.ds(0, QC)], czv, csem).wait()

    for s in (0, 1):
        _, _, dsts, _, _, ssem = sets[s]
        for q in range(2):
            pltpu.make_async_copy(
                dsts[q].at[:, pl.ds(0, C)],
                g2_hbm.at[pl.ds(q * QB, QB), pl.ds(0, C)],
                ssem).wait()


FB = 4
FW = FB * TB
EB = 8192
PARTS = (16, 56, 56, 52, 16)


def _expand_body(gft_ref, gfp_ref):
    gfp_ref[:, 0:C] = gft_ref[...].T
    gfp_ref[:, C:2 * C] = jnp.zeros((EB, C), jnp.float32)


def _fuse_body2(lf_ref, g2_ref, prev_ref, out_ref):
    del prev_ref
    _fuse_body(lf_ref, g2_ref, out_ref)


def _fuse_body(lf_ref, g2_ref, out_ref):
    for b in range(FB):
        lft = lf_ref[:, pl.ds(b * TB, TB)]
        g2t = g2_ref[pl.ds(b * H, H), :].T
        out_ref[:, pl.ds(b * TB, H)] = lft[:, 0:H] + g2t[0:C, :]
        out_ref[:, pl.ds(b * TB + H, H)] = lft[:, H:TB] + g2t[C:2 * C, :]


@jax.jit
def _fusion(cx, cy, cz, lft, gft):
    gfp = pl.pallas_call(
        _expand_body,
        grid=(N_GLOBAL // EB,),
        in_specs=[pl.BlockSpec((C, EB), lambda i: (0, i))],
        out_specs=pl.BlockSpec((EB, 2 * C), lambda i: (i, 0)),
        out_shape=jax.ShapeDtypeStruct((N_GLOBAL, 2 * C), jnp.float32),
    )(gft)

    mesh = plsc.VectorSubcoreMesh(core_axis_name="c", subcore_axis_name="s")

    def sc_gather(b0, nblocks):
        return pl.kernel(
            functools.partial(_gather_body, b0, 4 * nblocks),
            out_type=jax.ShapeDtypeStruct((nblocks * H, 2 * C), jnp.float32),
            mesh=mesh,
            scratch_types=(
                [pltpu.VMEM((QC,), jnp.int32)] * 6
                + [pltpu.VMEM((QB,), jnp.int32)] * 4
                + [pltpu.VMEM((QB, 2 * C), jnp.float32)] * 4
                + [pltpu.SemaphoreType.DMA] * 6
            ),
            compiler_params=pltpu.CompilerParams(
                use_tc_tiling_on_sc=False, needs_layout_passes=False),
        )(cx, cy, cz, gfp)

    g2s = []
    b0 = 0
    for nb in PARTS:
        g2s.append(sc_gather(b0, nb))
        b0 += nb

    out_t = None
    b0 = 0
    for p, nb in enumerate(PARTS):
        off = b0 // FB
        lf_spec = pl.BlockSpec((C, FW), lambda i, off=off: (0, i + off))
        g2_spec = pl.BlockSpec((FB * H, 2 * C), lambda i: (i, 0))
        out_spec = pl.BlockSpec((C, FW), lambda i, off=off: (0, i + off))
        out_shape = jax.ShapeDtypeStruct((C, N), jnp.float32)
        if p == 0:
            out_t = pl.pallas_call(
                _fuse_body,
                grid=(nb // FB,),
                in_specs=[lf_spec, g2_spec],
                out_specs=out_spec,
                out_shape=out_shape,
            )(lft, g2s[p])
        else:
            out_t = pl.pallas_call(
                _fuse_body2,
                grid=(nb // FB,),
                in_specs=[lf_spec, g2_spec,
                          pl.BlockSpec(memory_space=pl.ANY)],
                out_specs=out_spec,
                out_shape=out_shape,
                input_output_aliases={2: 0},
            )(lft, g2s[p], out_t)
        b0 += nb
    return out_t


def kernel(local_features, local_coords, local_base, global_features, global_base):
    adj = (local_coords.astype(jnp.int32)
           + local_base.astype(jnp.int32)[None, :]
           - SCALE * global_base.astype(jnp.int32)[None, :])
    pad = (0, NP - N)
    cx = jnp.pad(adj[:, 0], pad)
    cy = jnp.pad(adj[:, 1], pad)
    cz = jnp.pad(adj[:, 2], pad)
    out_t = _fusion(cx, cy, cz, local_features.T, global_features.T)
    return out_t.T
